# baseline (device time: 200417 ns/iter reference)
import jax
import jax.numpy as jnp
from jax import lax
from jax.experimental import pallas as pl
from jax.experimental.pallas import tpu as pltpu

N_Y = 4
B, S, D = 2, 512, 2048
H, DH, DR = 16, 128, 32
DC_SH = 128
HC = DC_SH // 2
BS = B * S
NCH = 4
CH = D // NCH
SCALE = (DH + DR) ** -0.5


def _mm(a, b, bn=1024):
    m, k = a.shape
    _, n = b.shape
    bn = min(bn, n)

    def body(a_ref, b_ref, o_ref):
        o_ref[:, :] = jnp.dot(a_ref[:, :], b_ref[:, :],
                              preferred_element_type=jnp.float32)

    return pl.pallas_call(
        body,
        grid=(n // bn,),
        in_specs=[
            pl.BlockSpec((m, k), lambda j: (0, 0)),
            pl.BlockSpec((k, bn), lambda j: (0, j)),
        ],
        out_specs=pl.BlockSpec((m, bn), lambda j: (0, j)),
        out_shape=jax.ShapeDtypeStruct((m, n), jnp.float32),
    )(a, b)


def _mm_tt(w, a):

    def body(w_ref, a_ref, o_ref):
        o_ref[:, :] = lax.dot_general(
            w_ref[:, :], a_ref[:, :], (((0,), (1,)), ((), ())),
            preferred_element_type=jnp.float32)

    k, n = w.shape
    m, _ = a.shape
    return pl.pallas_call(
        body,
        in_specs=[pl.BlockSpec(memory_space=pltpu.VMEM)] * 2,
        out_specs=pl.BlockSpec(memory_space=pltpu.VMEM),
        out_shape=jax.ShapeDtypeStruct((n, m), jnp.float32),
    )(w, a)


def _gather_kv(cT_sh, Wuk, Wuv):

    def body(cs_ref, wuk_ref, wuv_ref, k_ref, v_ref,
             c4_ref, wuk4_ref, wuv4_ref, fsend, frecv, bsend, brecv):
        xi = lax.axis_index("x")
        my = lax.axis_index("y")
        zi = lax.axis_index("z")
        left = lax.rem(my + N_Y - 1, N_Y)
        right = lax.rem(my + 1, N_Y)

        barrier = pltpu.get_barrier_semaphore()
        pl.semaphore_signal(barrier, inc=1, device_id=(xi, left, zi),
                            device_id_type=pl.DeviceIdType.MESH)
        pl.semaphore_signal(barrier, inc=1, device_id=(xi, right, zi),
                            device_id_type=pl.DeviceIdType.MESH)
        pl.semaphore_wait(barrier, 2)

        c4_ref[my] = cs_ref[:, :]
        wuk4_ref[my] = wuk_ref[:, :]
        wuv4_ref[my] = wuv_ref[:, :]

        def start(h, slot, target, send_sems, recv_sems, half):
            lo = half * HC
            srcs = (
                c4_ref.at[slot, pl.ds(lo, HC), :],
                wuk4_ref.at[slot, pl.ds(lo, HC), :],
                wuv4_ref.at[slot, pl.ds(lo, HC), :],
            )
            rdmas = []
            for t, src in enumerate(srcs):
                rdma = pltpu.make_async_remote_copy(
                    src_ref=src,
                    dst_ref=src,
                    send_sem=send_sems.at[t, h],
                    recv_sem=recv_sems.at[t, h],
                    device_id=(xi, target, zi),
                    device_id_type=pl.DeviceIdType.MESH,
                )
                rdma.start()
                rdmas.append(rdma)
            return rdmas

        def acc(slot, half, first=False):
            lo = half * HC
            c_half = c4_ref[slot, pl.ds(lo, HC), :]
            for jc in range(NCH):
                cols = pl.ds(jc * CH, CH)
                for out_ref, w4_ref in ((k_ref, wuk4_ref), (v_ref, wuv4_ref)):
                    w_half = w4_ref[slot, pl.ds(lo, HC), cols]
                    contrib = lax.dot_general(
                        c_half, w_half, (((0,), (0,)), ((), ())),
                        preferred_element_type=jnp.float32)
                    if first:
                        out_ref[:, cols] = contrib
                    else:
                        out_ref[:, cols] += contrib

        for h in range(N_Y - 1):
            fwd_slot = lax.rem(my + N_Y - h, N_Y)
            bwd_slot = lax.rem(my + h, N_Y)
            rdmas = start(h, fwd_slot, right, fsend, frecv, half=0)
            rdmas += start(h, bwd_slot, left, bsend, brecv, half=1)
            if h == 0:
                acc(my, 0, first=True)
                acc(my, 1)
            else:
                acc(lax.rem(my + N_Y - h, N_Y), 0)
                acc(lax.rem(my + h, N_Y), 1)
            for r in rdmas:
                r.wait()
        acc(lax.rem(my + 1, N_Y), 0)
        acc(lax.rem(my + N_Y - 1, N_Y), 1)

    return pl.pallas_call(
        body,
        in_specs=[pl.BlockSpec(memory_space=pltpu.VMEM)] * 3,
        out_specs=[pl.BlockSpec(memory_space=pltpu.VMEM)] * 2,
        out_shape=[
            jax.ShapeDtypeStruct((BS, D), jnp.float32),
            jax.ShapeDtypeStruct((BS, D), jnp.float32),
        ],
        scratch_shapes=[
            pltpu.VMEM((N_Y, DC_SH, BS), jnp.float32),
            pltpu.VMEM((N_Y, DC_SH, D), jnp.float32),
            pltpu.VMEM((N_Y, DC_SH, D), jnp.float32),
            pltpu.SemaphoreType.DMA((3, N_Y - 1)),
            pltpu.SemaphoreType.DMA((3, N_Y - 1)),
            pltpu.SemaphoreType.DMA((3, N_Y - 1)),
            pltpu.SemaphoreType.DMA((3, N_Y - 1)),
        ],
        compiler_params=pltpu.CompilerParams(collective_id=0),
    )(cT_sh, Wuk, Wuv)


def _attention_out(Q, K, V, QrT, Kr, Wo):

    def body(q_ref, k_ref, v_ref, qr_ref, kr_ref, wo_ref, o_ref):
        h = pl.program_id(1)
        q = q_ref[:, :]
        k = k_ref[:, :]
        v = v_ref[:, :]
        qr_t = qr_ref[:, :]
        kr = kr_ref[:, :]
        s = lax.dot_general(q, k, (((1,), (1,)), ((), ())),
                            preferred_element_type=jnp.float32)
        s = s + lax.dot_general(qr_t, kr, (((0,), (1,)), ((), ())),
                                preferred_element_type=jnp.float32)
        s = s * SCALE
        m = jnp.max(s, axis=-1, keepdims=True)
        p = jnp.exp(s - m)
        p = p / jnp.sum(p, axis=-1, keepdims=True)
        o_h = jnp.dot(p, v, preferred_element_type=jnp.float32)
        contrib = jnp.dot(o_h, wo_ref[:, :],
                          preferred_element_type=jnp.float32)

        @pl.when(h == 0)
        def _():
            o_ref[:, :] = contrib

        @pl.when(h != 0)
        def _():
            o_ref[:, :] += contrib

    return pl.pallas_call(
        body,
        grid=(B, H),
        in_specs=[
            pl.BlockSpec((S, DH), lambda b, h: (b, h)),
            pl.BlockSpec((S, DH), lambda b, h: (b, h)),
            pl.BlockSpec((S, DH), lambda b, h: (b, h)),
            pl.BlockSpec((DR, S), lambda b, h: (h, b)),
            pl.BlockSpec((S, DR), lambda b, h: (b, 0)),
            pl.BlockSpec((DH, D), lambda b, h: (h, 0)),
        ],
        out_specs=pl.BlockSpec((S, D), lambda b, h: (b, 0)),
        out_shape=jax.ShapeDtypeStruct((BS, D), jnp.float32),
    )(Q, K, V, QrT, Kr, Wo)


def kernel(x, Wdkv, Wuk, Wuv, Wq, Wqr, Wkr, Wo):
    x2 = x.reshape(BS, D)
    cT = _mm_tt(Wdkv, x2)
    K, V = _gather_kv(cT, Wuk, Wuv)
    Q = _mm(x2, Wq)
    QrT = _mm_tt(Wqr, x2)
    Kr = _mm(x2, Wkr, bn=DR)
    out = _attention_out(Q, K, V, QrT, Kr, Wo)
    return out.reshape(B, S, D)
